# baseline (speedup 1.0000x reference)
import functools

import jax
import jax.numpy as jnp
from jax import lax
from jax.experimental import pallas as pl
from jax.experimental.pallas import tpu as pltpu
from jax.experimental.pallas import tpu_sc as plsc

B = 16384
SPARSE_VOCAB = 1000
VOCAB_PAD = 1024
EMB_DIM = 64
HID = 50
OUT = 50
OUT_PAD = 64

NC = 2
NS = 16
NW = NC * NS
B_PER_W = B // NW
CHUNK = 128
N_CHUNK = B_PER_W // CHUNK


def _precompute_body(ws1_ref, bs1_ref, ws2_ref, bs2_ref, t_ref):
    h = jnp.maximum(ws1_ref[...] + bs1_ref[...], 0.0)
    t_ref[...] = (
        jnp.dot(h, ws2_ref[...], preferred_element_type=jnp.float32)
        + bs2_ref[...]
    )


def _precompute_table(ws1p, bs1, ws2p, bs2p):
    return pl.pallas_call(
        _precompute_body,
        out_shape=jax.ShapeDtypeStruct((VOCAB_PAD, OUT_PAD), jnp.float32),
    )(ws1p, bs1, ws2p, bs2p)


def _sc_gather_body(emb_hbm, didx_hbm, t_hbm, sidx_hbm,
                    demb_out, srow_out,
                    didx_v, sidx_v, drows_v, srows_v, dsem, ssem):
    wid = lax.axis_index("s") * NC + lax.axis_index("c")
    base = wid * B_PER_W
    row0 = wid * N_CHUNK
    pltpu.sync_copy(didx_hbm.at[pl.ds(row0, N_CHUNK)], didx_v)
    pltpu.sync_copy(sidx_hbm.at[pl.ds(row0, N_CHUNK)], sidx_v)
    copies = []
    for j in range(N_CHUNK):
        copies.append(pltpu.async_copy(
            emb_hbm.at[didx_v.at[j]],
            drows_v.at[pl.ds(j * CHUNK, CHUNK)], dsem))
        copies.append(pltpu.async_copy(
            t_hbm.at[sidx_v.at[j]],
            srows_v.at[pl.ds(j * CHUNK, CHUNK)], ssem))
    for cp in copies:
        cp.wait()
    pltpu.sync_copy(drows_v, demb_out.at[pl.ds(base, B_PER_W)])
    pltpu.sync_copy(srows_v, srow_out.at[pl.ds(base, B_PER_W)])


def _sc_gather(emb_table, didx2d, t_table, sidx2d):
    mesh = plsc.VectorSubcoreMesh(core_axis_name="c", subcore_axis_name="s")
    return pl.kernel(
        _sc_gather_body,
        mesh=mesh,
        out_type=[
            jax.ShapeDtypeStruct((B, EMB_DIM), jnp.float32),
            jax.ShapeDtypeStruct((B, OUT_PAD), jnp.float32),
        ],
        scratch_types=[
            pltpu.VMEM((N_CHUNK, CHUNK), jnp.int32),
            pltpu.VMEM((N_CHUNK, CHUNK), jnp.int32),
            pltpu.VMEM((B_PER_W, EMB_DIM), jnp.float32),
            pltpu.VMEM((B_PER_W, OUT_PAD), jnp.float32),
            pltpu.SemaphoreType.DMA,
            pltpu.SemaphoreType.DMA,
        ],
        compiler_params=pltpu.CompilerParams(use_tc_tiling_on_sc=False),
    )(emb_table, didx2d, t_table, sidx2d)


_MLP_BLOCK = 2048


def _mlp_body(sg_ref, de_ref, wd1_ref, bd1_ref, wd2_ref, bd2_ref,
              sout_ref, dout_ref):
    sout_ref[...] = sg_ref[:, :OUT]
    h = jnp.maximum(
        jnp.dot(de_ref[...], wd1_ref[...], preferred_element_type=jnp.float32)
        + bd1_ref[...], 0.0)
    dout_ref[...] = (
        jnp.dot(h, wd2_ref[...], preferred_element_type=jnp.float32)
        + bd2_ref[...]
    )


def _mlp(s_rows, d_emb, wd1, bd1, wd2, bd2):
    nblk = B // _MLP_BLOCK
    return pl.pallas_call(
        _mlp_body,
        grid=(nblk,),
        in_specs=[
            pl.BlockSpec((_MLP_BLOCK, OUT_PAD), lambda i: (i, 0)),
            pl.BlockSpec((_MLP_BLOCK, EMB_DIM), lambda i: (i, 0)),
            pl.BlockSpec((EMB_DIM, HID), lambda i: (0, 0)),
            pl.BlockSpec((1, HID), lambda i: (0, 0)),
            pl.BlockSpec((HID, OUT), lambda i: (0, 0)),
            pl.BlockSpec((1, OUT), lambda i: (0, 0)),
        ],
        out_specs=[
            pl.BlockSpec((_MLP_BLOCK, OUT), lambda i: (i, 0)),
            pl.BlockSpec((_MLP_BLOCK, OUT), lambda i: (i, 0)),
        ],
        out_shape=[
            jax.ShapeDtypeStruct((B, OUT), jnp.float32),
            jax.ShapeDtypeStruct((B, OUT), jnp.float32),
        ],
    )(s_rows, d_emb, wd1, bd1, wd2, bd2)


def kernel(sparse_col_inp, dense_col_inp, emb_table, Ws1, bs1, Ws2, bs2,
           Wd1, bd1, Wd2, bd2):
    sidx = sparse_col_inp.astype(jnp.int32).reshape(B // CHUNK, CHUNK)
    didx = dense_col_inp.astype(jnp.int32).reshape(B // CHUNK, CHUNK)
    ws1p = jnp.pad(Ws1, ((0, VOCAB_PAD - SPARSE_VOCAB), (0, 0)))
    ws2p = jnp.pad(Ws2, ((0, 0), (0, OUT_PAD - OUT)))
    bs2p = jnp.pad(bs2, (0, OUT_PAD - OUT))
    t_table = _precompute_table(ws1p, bs1.reshape(1, HID),
                                ws2p, bs2p.reshape(1, OUT_PAD))
    d_emb, s_rows = _sc_gather(emb_table, didx, t_table, sidx)
    sparse_out, dense_out = _mlp(s_rows, d_emb, Wd1, bd1.reshape(1, HID),
                                 Wd2, bd2.reshape(1, OUT))
    return (sparse_out, dense_out)
